# baseline (speedup 1.0000x reference)
import functools

import jax
import jax.numpy as jnp
from jax.experimental import pallas as pl
from jax.experimental.pallas import tpu as pltpu

_BN_EPS = 1e-3
_EPS_SCALE = 0.01
_G = 8


def _dot(a, b):
    return jax.lax.dot_general(
        a, b, (((1,), (0,)), ((), ())), preferred_element_type=jnp.float32
    )


def _fold_bn(bn):
    s = bn["gamma"] * jax.lax.rsqrt(bn["var"] + _BN_EPS)
    t = bn["beta"] - bn["mean"] * s
    return s, t


def _body(n_nodes, kl_scale,
          x_ref, adj_ref, eps_ref,
          w1, b1, w2, b2, w3, b3, w4, b4, wf, bf,
          wm, bm, wv, bv,
          wo1, bo1, wo2, bo2, wo3, bo3,
          wa1, ba1, wa2, ba2, wa3, ba3,
          clsv, dmat, ebias,
          ops_out, padj_out, kl_out, mean_out,
          agg_scr):
    relu = jax.nn.relu
    N = n_nodes
    G = _G
    rows = G * N
    for g in range(G):
        xg = x_ref[g]
        agg_scr[g * N:(g + 1) * N, :] = xg + _dot(adj_ref[g], xg)
    h = relu(_dot(agg_scr[...], w1[...]) + b1[...])
    h = relu(_dot(h, w2[...]) + b2[...])
    h = relu(_dot(h, w3[...]) + b3[...])
    h = relu(_dot(h, w4[...]) + b4[...])
    h = relu(_dot(h, wf[...]) + bf[...])
    mean = _dot(h, wm[...]) + bm[...]
    var = _dot(h, wv[...]) + bv[...]
    latent = mean.shape[1]
    mean_out[...] = mean.reshape(G, N, latent)
    evar = jnp.exp(var)
    kl_out[...] = (jnp.sum(1.0 + var - mean * mean - evar) * kl_scale).reshape(
        1, 1, 1)
    eps = eps_ref[...].reshape(rows, latent)
    c = mean + jnp.exp(var * 0.5) * eps * _EPS_SCALE
    o = relu(_dot(c, wo1[...]) + bo1[...])
    o = relu(_dot(o, wo2[...]) + bo2[...])
    lo = _dot(o, wo3[...]) + bo3[...]
    m = jnp.max(lo, axis=-1, keepdims=True)
    e = jnp.exp(lo - m)
    sm = e / jnp.sum(e, axis=-1, keepdims=True)
    ops_out[...] = sm.reshape(G, N, lo.shape[1])
    a = relu(_dot(c, wa1[...]) + ba1[...])
    a = relu(_dot(a, wa2[...]) + ba2[...])
    a = relu(_dot(a, wa3[...]) + ba3[...])
    s = jax.nn.sigmoid(a * clsv[0, 0] + clsv[0, 1])
    padj = _dot(s, dmat[...]) + ebias[...]
    padj_out[...] = padj.reshape(G, N, 2 * N)


def kernel(x, adj, eps_noise, params):
    B, N, F = x.shape
    p = params
    latent = p["Wm"].shape[1]
    num_ops = p["ops_W"][2].shape[1]
    f32 = jnp.float32

    s0, t0 = _fold_bn(p["gin_bn"][0])
    s1, t1 = _fold_bn(p["gin_bn"][1])
    s2, t2 = _fold_bn(p["gin_bn"][2])
    s3, t3 = _fold_bn(p["gin_bn"][3])
    se, te = _fold_bn(p["enc_bn"])
    w1, b1 = p["gin_W"][0], p["gin_b"][0]
    w2 = s0[:, None] * p["gin_W"][1]
    b2 = p["gin_b"][1] + t0 @ p["gin_W"][1]
    w3 = s1[:, None] * p["gin_W"][2]
    b3 = p["gin_b"][2] + t1 @ p["gin_W"][2]
    w4 = s2[:, None] * p["gin_W"][3]
    b4 = p["gin_b"][3] + t2 @ p["gin_W"][3]
    wf = s3[:, None] * p["gin_Wf"]
    bf = p["gin_bf"] + t3 @ p["gin_Wf"]
    wm = se[:, None] * p["Wm"]
    bm = p["bm"] + te @ p["Wm"]
    wv = se[:, None] * p["Wv"]
    bv = p["bv"] + te @ p["Wv"]

    dw = p["cls_W"][0, 1] - p["cls_W"][0, 0]
    db = p["cls_b"][1] - p["cls_b"][0]
    clsv = jnp.stack([dw, db]).reshape(1, 2)
    j = jnp.arange(N)
    k = jnp.arange(2 * N)
    dmat = (jnp.where(k[None, :] == 2 * j[:, None], -1.0, 0.0)
            + jnp.where(k[None, :] == 2 * j[:, None] + 1, 1.0, 0.0)).astype(f32)
    ebias = (1.0 - (k % 2)).astype(f32).reshape(1, 2 * N)

    nb = B // _G
    const2 = lambda i: (0, 0)
    wspec = lambda a: pl.BlockSpec(a.shape, const2)
    blk3 = lambda d: pl.BlockSpec((_G, N, d), lambda i: (i, 0, 0))

    b1r, b2r, b3r, b4r, bfr = (v.reshape(1, -1) for v in (b1, b2, b3, b4, bf))
    bmr, bvr = bm.reshape(1, -1), bv.reshape(1, -1)
    bo1, bo2, bo3 = (v.reshape(1, -1) for v in p["ops_b"])
    ba1, ba2, ba3 = (v.reshape(1, -1) for v in p["adj_b"])
    wo1, wo2, wo3 = p["ops_W"]
    wa1, wa2, wa3 = p["adj_W"]

    weight_args = [w1, b1r, w2, b2r, w3, b3r, w4, b4r, wf, bfr,
                   wm, bmr, wv, bvr,
                   wo1, bo1, wo2, bo2, wo3, bo3,
                   wa1, ba1, wa2, ba2, wa3, ba3,
                   clsv, dmat, ebias]

    out = pl.pallas_call(
        functools.partial(_body, N, -0.5 / float(B * N)),
        grid=(nb,),
        in_specs=[
            blk3(F),
            pl.BlockSpec((_G, N, N), lambda i: (i, 0, 0)),
            blk3(latent),
        ] + [wspec(a) for a in weight_args],
        out_specs=[
            blk3(num_ops),
            pl.BlockSpec((_G, N, 2 * N), lambda i: (i, 0, 0)),
            pl.BlockSpec((1, 1, 1), lambda i: (i, 0, 0)),
            blk3(latent),
        ],
        out_shape=[
            jax.ShapeDtypeStruct((B, N, num_ops), f32),
            jax.ShapeDtypeStruct((B, N, 2 * N), f32),
            jax.ShapeDtypeStruct((nb, 1, 1), f32),
            jax.ShapeDtypeStruct((B, N, latent), f32),
        ],
        scratch_shapes=[
            pltpu.VMEM((_G * N, F), f32),
        ],
        compiler_params=pltpu.CompilerParams(
            dimension_semantics=("arbitrary",),
        ),
    )(x, adj, eps_noise, *weight_args)

    ops_cls, padj, klp, mean = out
    adj_cls = padj.reshape(B, N * N, 2)
    return ops_cls, adj_cls, jnp.sum(klp), mean


# speedup vs baseline: 1.2730x; 1.2730x over previous
import functools

import jax
import jax.numpy as jnp
from jax.experimental import pallas as pl
from jax.experimental.pallas import tpu as pltpu

_BN_EPS = 1e-3
_EPS_SCALE = 0.01
_G = 16


def _dot(a, b):
    return jax.lax.dot_general(
        a, b, (((1,), (0,)), ((), ())), preferred_element_type=jnp.float32
    )


def _fold_bn(bn):
    s = bn["gamma"] * jax.lax.rsqrt(bn["var"] + _BN_EPS)
    t = bn["beta"] - bn["mean"] * s
    return s, t


def _body(n_nodes, kl_scale,
          x_ref, adj_ref, eps_ref,
          w1, b1, w2, b2, w3, b3, w4, b4, wf, bf,
          wm, bm, wv, bv,
          wo1, bo1, wo2, bo2, wo3, bo3,
          wa1, ba1, wa2, ba2, wa3, ba3,
          clsv, dmat, ebias,
          ops_out, padj_out, kl_out, mean_out,
          agg_scr):
    relu = jax.nn.relu
    N = n_nodes
    G = _G
    rows = G * N
    for g in range(G):
        xg = x_ref[g]
        agg_scr[g * N:(g + 1) * N, :] = xg + _dot(adj_ref[g], xg)
    h = relu(_dot(agg_scr[...], w1[...]) + b1[...])
    h = relu(_dot(h, w2[...]) + b2[...])
    h = relu(_dot(h, w3[...]) + b3[...])
    h = relu(_dot(h, w4[...]) + b4[...])
    h = relu(_dot(h, wf[...]) + bf[...])
    mean = _dot(h, wm[...]) + bm[...]
    var = _dot(h, wv[...]) + bv[...]
    latent = mean.shape[1]
    mean_out[...] = mean.reshape(G, N, latent)
    evar = jnp.exp(var)
    kl_out[...] = (jnp.sum(1.0 + var - mean * mean - evar) * kl_scale).reshape(
        1, 1, 1)
    eps = eps_ref[...].reshape(rows, latent)
    c = mean + jnp.exp(var * 0.5) * eps * _EPS_SCALE
    o = relu(_dot(c, wo1[...]) + bo1[...])
    o = relu(_dot(o, wo2[...]) + bo2[...])
    lo = _dot(o, wo3[...]) + bo3[...]
    m = jnp.max(lo, axis=-1, keepdims=True)
    e = jnp.exp(lo - m)
    sm = e / jnp.sum(e, axis=-1, keepdims=True)
    ops_out[...] = sm.reshape(G, N, lo.shape[1])
    a = relu(_dot(c, wa1[...]) + ba1[...])
    a = relu(_dot(a, wa2[...]) + ba2[...])
    a = relu(_dot(a, wa3[...]) + ba3[...])
    s = jax.nn.sigmoid(a * clsv[0, 0] + clsv[0, 1])
    padj = _dot(s, dmat[...]) + ebias[...]
    padj_out[...] = padj.reshape(G, N, 2 * N)


def kernel(x, adj, eps_noise, params):
    B, N, F = x.shape
    p = params
    latent = p["Wm"].shape[1]
    num_ops = p["ops_W"][2].shape[1]
    f32 = jnp.float32

    s0, t0 = _fold_bn(p["gin_bn"][0])
    s1, t1 = _fold_bn(p["gin_bn"][1])
    s2, t2 = _fold_bn(p["gin_bn"][2])
    s3, t3 = _fold_bn(p["gin_bn"][3])
    se, te = _fold_bn(p["enc_bn"])
    w1, b1 = p["gin_W"][0], p["gin_b"][0]
    w2 = s0[:, None] * p["gin_W"][1]
    b2 = p["gin_b"][1] + t0 @ p["gin_W"][1]
    w3 = s1[:, None] * p["gin_W"][2]
    b3 = p["gin_b"][2] + t1 @ p["gin_W"][2]
    w4 = s2[:, None] * p["gin_W"][3]
    b4 = p["gin_b"][3] + t2 @ p["gin_W"][3]
    wf = s3[:, None] * p["gin_Wf"]
    bf = p["gin_bf"] + t3 @ p["gin_Wf"]
    wm = se[:, None] * p["Wm"]
    bm = p["bm"] + te @ p["Wm"]
    wv = se[:, None] * p["Wv"]
    bv = p["bv"] + te @ p["Wv"]

    dw = p["cls_W"][0, 1] - p["cls_W"][0, 0]
    db = p["cls_b"][1] - p["cls_b"][0]
    clsv = jnp.stack([dw, db]).reshape(1, 2)
    j = jnp.arange(N)
    k = jnp.arange(2 * N)
    dmat = (jnp.where(k[None, :] == 2 * j[:, None], -1.0, 0.0)
            + jnp.where(k[None, :] == 2 * j[:, None] + 1, 1.0, 0.0)).astype(f32)
    ebias = (1.0 - (k % 2)).astype(f32).reshape(1, 2 * N)

    nb = B // _G
    const2 = lambda i: (0, 0)
    wspec = lambda a: pl.BlockSpec(a.shape, const2)
    blk3 = lambda d: pl.BlockSpec((_G, N, d), lambda i: (i, 0, 0))

    b1r, b2r, b3r, b4r, bfr = (v.reshape(1, -1) for v in (b1, b2, b3, b4, bf))
    bmr, bvr = bm.reshape(1, -1), bv.reshape(1, -1)
    bo1, bo2, bo3 = (v.reshape(1, -1) for v in p["ops_b"])
    ba1, ba2, ba3 = (v.reshape(1, -1) for v in p["adj_b"])
    wo1, wo2, wo3 = p["ops_W"]
    wa1, wa2, wa3 = p["adj_W"]

    weight_args = [w1, b1r, w2, b2r, w3, b3r, w4, b4r, wf, bfr,
                   wm, bmr, wv, bvr,
                   wo1, bo1, wo2, bo2, wo3, bo3,
                   wa1, ba1, wa2, ba2, wa3, ba3,
                   clsv, dmat, ebias]

    out = pl.pallas_call(
        functools.partial(_body, N, -0.5 / float(B * N)),
        grid=(nb,),
        in_specs=[
            blk3(F),
            pl.BlockSpec((_G, N, N), lambda i: (i, 0, 0)),
            blk3(latent),
        ] + [wspec(a) for a in weight_args],
        out_specs=[
            blk3(num_ops),
            pl.BlockSpec((_G, N, 2 * N), lambda i: (i, 0, 0)),
            pl.BlockSpec((1, 1, 1), lambda i: (i, 0, 0)),
            blk3(latent),
        ],
        out_shape=[
            jax.ShapeDtypeStruct((B, N, num_ops), f32),
            jax.ShapeDtypeStruct((B, N, 2 * N), f32),
            jax.ShapeDtypeStruct((nb, 1, 1), f32),
            jax.ShapeDtypeStruct((B, N, latent), f32),
        ],
        scratch_shapes=[
            pltpu.VMEM((_G * N, F), f32),
        ],
        compiler_params=pltpu.CompilerParams(
            dimension_semantics=("arbitrary",),
        ),
    )(x, adj, eps_noise, *weight_args)

    ops_cls, padj, klp, mean = out
    adj_cls = padj.reshape(B, N * N, 2)
    return ops_cls, adj_cls, jnp.sum(klp), mean


# speedup vs baseline: 1.4337x; 1.1262x over previous
import functools

import jax
import jax.numpy as jnp
from jax.experimental import pallas as pl
from jax.experimental.pallas import tpu as pltpu

_BN_EPS = 1e-3
_EPS_SCALE = 0.01
_G = 32


def _dot(a, b):
    return jax.lax.dot_general(
        a, b, (((1,), (0,)), ((), ())), preferred_element_type=jnp.float32
    )


def _fold_bn(bn):
    s = bn["gamma"] * jax.lax.rsqrt(bn["var"] + _BN_EPS)
    t = bn["beta"] - bn["mean"] * s
    return s, t


def _body(n_nodes, kl_scale,
          x_ref, adj_ref, eps_ref,
          w1, b1, w2, b2, w3, b3, w4, b4, wf, bf,
          wm, bm, wv, bv,
          wo1, bo1, wo2, bo2, wo3, bo3,
          wa1, ba1, wa2, ba2, wa3, ba3,
          clsv, dmat, ebias,
          ops_out, padj_out, kl_out, mean_out,
          agg_scr):
    relu = jax.nn.relu
    N = n_nodes
    G = _G
    rows = G * N
    for g in range(G):
        xg = x_ref[g]
        agg_scr[g * N:(g + 1) * N, :] = xg + _dot(adj_ref[g], xg)
    h = relu(_dot(agg_scr[...], w1[...]) + b1[...])
    h = relu(_dot(h, w2[...]) + b2[...])
    h = relu(_dot(h, w3[...]) + b3[...])
    h = relu(_dot(h, w4[...]) + b4[...])
    h = relu(_dot(h, wf[...]) + bf[...])
    mean = _dot(h, wm[...]) + bm[...]
    var = _dot(h, wv[...]) + bv[...]
    latent = mean.shape[1]
    mean_out[...] = mean.reshape(G, N, latent)
    evar = jnp.exp(var)
    kl_out[...] = (jnp.sum(1.0 + var - mean * mean - evar) * kl_scale).reshape(
        1, 1, 1)
    eps = eps_ref[...].reshape(rows, latent)
    c = mean + jnp.exp(var * 0.5) * eps * _EPS_SCALE
    o = relu(_dot(c, wo1[...]) + bo1[...])
    o = relu(_dot(o, wo2[...]) + bo2[...])
    lo = _dot(o, wo3[...]) + bo3[...]
    m = jnp.max(lo, axis=-1, keepdims=True)
    e = jnp.exp(lo - m)
    sm = e / jnp.sum(e, axis=-1, keepdims=True)
    ops_out[...] = sm.reshape(G, N, lo.shape[1])
    a = relu(_dot(c, wa1[...]) + ba1[...])
    a = relu(_dot(a, wa2[...]) + ba2[...])
    a = relu(_dot(a, wa3[...]) + ba3[...])
    s = jax.nn.sigmoid(a * clsv[0, 0] + clsv[0, 1])
    padj = _dot(s, dmat[...]) + ebias[...]
    padj_out[...] = padj.reshape(G, N, 2 * N)


def kernel(x, adj, eps_noise, params):
    B, N, F = x.shape
    p = params
    latent = p["Wm"].shape[1]
    num_ops = p["ops_W"][2].shape[1]
    f32 = jnp.float32

    s0, t0 = _fold_bn(p["gin_bn"][0])
    s1, t1 = _fold_bn(p["gin_bn"][1])
    s2, t2 = _fold_bn(p["gin_bn"][2])
    s3, t3 = _fold_bn(p["gin_bn"][3])
    se, te = _fold_bn(p["enc_bn"])
    w1, b1 = p["gin_W"][0], p["gin_b"][0]
    w2 = s0[:, None] * p["gin_W"][1]
    b2 = p["gin_b"][1] + t0 @ p["gin_W"][1]
    w3 = s1[:, None] * p["gin_W"][2]
    b3 = p["gin_b"][2] + t1 @ p["gin_W"][2]
    w4 = s2[:, None] * p["gin_W"][3]
    b4 = p["gin_b"][3] + t2 @ p["gin_W"][3]
    wf = s3[:, None] * p["gin_Wf"]
    bf = p["gin_bf"] + t3 @ p["gin_Wf"]
    wm = se[:, None] * p["Wm"]
    bm = p["bm"] + te @ p["Wm"]
    wv = se[:, None] * p["Wv"]
    bv = p["bv"] + te @ p["Wv"]

    dw = p["cls_W"][0, 1] - p["cls_W"][0, 0]
    db = p["cls_b"][1] - p["cls_b"][0]
    clsv = jnp.stack([dw, db]).reshape(1, 2)
    j = jnp.arange(N)
    k = jnp.arange(2 * N)
    dmat = (jnp.where(k[None, :] == 2 * j[:, None], -1.0, 0.0)
            + jnp.where(k[None, :] == 2 * j[:, None] + 1, 1.0, 0.0)).astype(f32)
    ebias = (1.0 - (k % 2)).astype(f32).reshape(1, 2 * N)

    nb = B // _G
    const2 = lambda i: (0, 0)
    wspec = lambda a: pl.BlockSpec(a.shape, const2)
    blk3 = lambda d: pl.BlockSpec((_G, N, d), lambda i: (i, 0, 0))

    b1r, b2r, b3r, b4r, bfr = (v.reshape(1, -1) for v in (b1, b2, b3, b4, bf))
    bmr, bvr = bm.reshape(1, -1), bv.reshape(1, -1)
    bo1, bo2, bo3 = (v.reshape(1, -1) for v in p["ops_b"])
    ba1, ba2, ba3 = (v.reshape(1, -1) for v in p["adj_b"])
    wo1, wo2, wo3 = p["ops_W"]
    wa1, wa2, wa3 = p["adj_W"]

    weight_args = [w1, b1r, w2, b2r, w3, b3r, w4, b4r, wf, bfr,
                   wm, bmr, wv, bvr,
                   wo1, bo1, wo2, bo2, wo3, bo3,
                   wa1, ba1, wa2, ba2, wa3, ba3,
                   clsv, dmat, ebias]

    out = pl.pallas_call(
        functools.partial(_body, N, -0.5 / float(B * N)),
        grid=(nb,),
        in_specs=[
            blk3(F),
            pl.BlockSpec((_G, N, N), lambda i: (i, 0, 0)),
            blk3(latent),
        ] + [wspec(a) for a in weight_args],
        out_specs=[
            blk3(num_ops),
            pl.BlockSpec((_G, N, 2 * N), lambda i: (i, 0, 0)),
            pl.BlockSpec((1, 1, 1), lambda i: (i, 0, 0)),
            blk3(latent),
        ],
        out_shape=[
            jax.ShapeDtypeStruct((B, N, num_ops), f32),
            jax.ShapeDtypeStruct((B, N, 2 * N), f32),
            jax.ShapeDtypeStruct((nb, 1, 1), f32),
            jax.ShapeDtypeStruct((B, N, latent), f32),
        ],
        scratch_shapes=[
            pltpu.VMEM((_G * N, F), f32),
        ],
        compiler_params=pltpu.CompilerParams(
            dimension_semantics=("arbitrary",),
        ),
    )(x, adj, eps_noise, *weight_args)

    ops_cls, padj, klp, mean = out
    adj_cls = padj.reshape(B, N * N, 2)
    return ops_cls, adj_cls, jnp.sum(klp), mean


# speedup vs baseline: 1.4623x; 1.0200x over previous
import functools

import jax
import jax.numpy as jnp
from jax.experimental import pallas as pl
from jax.experimental.pallas import tpu as pltpu

_BN_EPS = 1e-3
_EPS_SCALE = 0.01
_G = 64


def _dot(a, b):
    return jax.lax.dot_general(
        a, b, (((1,), (0,)), ((), ())), preferred_element_type=jnp.float32
    )


def _fold_bn(bn):
    s = bn["gamma"] * jax.lax.rsqrt(bn["var"] + _BN_EPS)
    t = bn["beta"] - bn["mean"] * s
    return s, t


def _body(n_nodes, kl_scale,
          x_ref, adj_ref, eps_ref,
          w1, b1, w2, b2, w3, b3, w4, b4, wf, bf,
          wm, bm, wv, bv,
          wo1, bo1, wo2, bo2, wo3, bo3,
          wa1, ba1, wa2, ba2, wa3, ba3,
          clsv, dmat, ebias,
          ops_out, padj_out, kl_out, mean_out,
          agg_scr):
    relu = jax.nn.relu
    N = n_nodes
    G = _G
    rows = G * N
    for g in range(G):
        xg = x_ref[g]
        agg_scr[g * N:(g + 1) * N, :] = xg + _dot(adj_ref[g], xg)
    h = relu(_dot(agg_scr[...], w1[...]) + b1[...])
    h = relu(_dot(h, w2[...]) + b2[...])
    h = relu(_dot(h, w3[...]) + b3[...])
    h = relu(_dot(h, w4[...]) + b4[...])
    h = relu(_dot(h, wf[...]) + bf[...])
    mean = _dot(h, wm[...]) + bm[...]
    var = _dot(h, wv[...]) + bv[...]
    latent = mean.shape[1]
    mean_out[...] = mean.reshape(G, N, latent)
    evar = jnp.exp(var)
    kl_out[...] = (jnp.sum(1.0 + var - mean * mean - evar) * kl_scale).reshape(
        1, 1, 1)
    eps = eps_ref[...].reshape(rows, latent)
    c = mean + jnp.exp(var * 0.5) * eps * _EPS_SCALE
    o = relu(_dot(c, wo1[...]) + bo1[...])
    o = relu(_dot(o, wo2[...]) + bo2[...])
    lo = _dot(o, wo3[...]) + bo3[...]
    m = jnp.max(lo, axis=-1, keepdims=True)
    e = jnp.exp(lo - m)
    sm = e / jnp.sum(e, axis=-1, keepdims=True)
    ops_out[...] = sm.reshape(G, N, lo.shape[1])
    a = relu(_dot(c, wa1[...]) + ba1[...])
    a = relu(_dot(a, wa2[...]) + ba2[...])
    a = relu(_dot(a, wa3[...]) + ba3[...])
    s = jax.nn.sigmoid(a * clsv[0, 0] + clsv[0, 1])
    padj = _dot(s, dmat[...]) + ebias[...]
    padj_out[...] = padj.reshape(G, N, 2 * N)


def kernel(x, adj, eps_noise, params):
    B, N, F = x.shape
    p = params
    latent = p["Wm"].shape[1]
    num_ops = p["ops_W"][2].shape[1]
    f32 = jnp.float32

    s0, t0 = _fold_bn(p["gin_bn"][0])
    s1, t1 = _fold_bn(p["gin_bn"][1])
    s2, t2 = _fold_bn(p["gin_bn"][2])
    s3, t3 = _fold_bn(p["gin_bn"][3])
    se, te = _fold_bn(p["enc_bn"])
    w1, b1 = p["gin_W"][0], p["gin_b"][0]
    w2 = s0[:, None] * p["gin_W"][1]
    b2 = p["gin_b"][1] + t0 @ p["gin_W"][1]
    w3 = s1[:, None] * p["gin_W"][2]
    b3 = p["gin_b"][2] + t1 @ p["gin_W"][2]
    w4 = s2[:, None] * p["gin_W"][3]
    b4 = p["gin_b"][3] + t2 @ p["gin_W"][3]
    wf = s3[:, None] * p["gin_Wf"]
    bf = p["gin_bf"] + t3 @ p["gin_Wf"]
    wm = se[:, None] * p["Wm"]
    bm = p["bm"] + te @ p["Wm"]
    wv = se[:, None] * p["Wv"]
    bv = p["bv"] + te @ p["Wv"]

    dw = p["cls_W"][0, 1] - p["cls_W"][0, 0]
    db = p["cls_b"][1] - p["cls_b"][0]
    clsv = jnp.stack([dw, db]).reshape(1, 2)
    j = jnp.arange(N)
    k = jnp.arange(2 * N)
    dmat = (jnp.where(k[None, :] == 2 * j[:, None], -1.0, 0.0)
            + jnp.where(k[None, :] == 2 * j[:, None] + 1, 1.0, 0.0)).astype(f32)
    ebias = (1.0 - (k % 2)).astype(f32).reshape(1, 2 * N)

    nb = B // _G
    const2 = lambda i: (0, 0)
    wspec = lambda a: pl.BlockSpec(a.shape, const2)
    blk3 = lambda d: pl.BlockSpec((_G, N, d), lambda i: (i, 0, 0))

    b1r, b2r, b3r, b4r, bfr = (v.reshape(1, -1) for v in (b1, b2, b3, b4, bf))
    bmr, bvr = bm.reshape(1, -1), bv.reshape(1, -1)
    bo1, bo2, bo3 = (v.reshape(1, -1) for v in p["ops_b"])
    ba1, ba2, ba3 = (v.reshape(1, -1) for v in p["adj_b"])
    wo1, wo2, wo3 = p["ops_W"]
    wa1, wa2, wa3 = p["adj_W"]

    weight_args = [w1, b1r, w2, b2r, w3, b3r, w4, b4r, wf, bfr,
                   wm, bmr, wv, bvr,
                   wo1, bo1, wo2, bo2, wo3, bo3,
                   wa1, ba1, wa2, ba2, wa3, ba3,
                   clsv, dmat, ebias]

    out = pl.pallas_call(
        functools.partial(_body, N, -0.5 / float(B * N)),
        grid=(nb,),
        in_specs=[
            blk3(F),
            pl.BlockSpec((_G, N, N), lambda i: (i, 0, 0)),
            blk3(latent),
        ] + [wspec(a) for a in weight_args],
        out_specs=[
            blk3(num_ops),
            pl.BlockSpec((_G, N, 2 * N), lambda i: (i, 0, 0)),
            pl.BlockSpec((1, 1, 1), lambda i: (i, 0, 0)),
            blk3(latent),
        ],
        out_shape=[
            jax.ShapeDtypeStruct((B, N, num_ops), f32),
            jax.ShapeDtypeStruct((B, N, 2 * N), f32),
            jax.ShapeDtypeStruct((nb, 1, 1), f32),
            jax.ShapeDtypeStruct((B, N, latent), f32),
        ],
        scratch_shapes=[
            pltpu.VMEM((_G * N, F), f32),
        ],
        compiler_params=pltpu.CompilerParams(
            dimension_semantics=("arbitrary",),
        ),
    )(x, adj, eps_noise, *weight_args)

    ops_cls, padj, klp, mean = out
    adj_cls = padj.reshape(B, N * N, 2)
    return ops_cls, adj_cls, jnp.sum(klp), mean
